# baseline (speedup 1.0000x reference)
import functools

import jax
import jax.numpy as jnp
from jax import lax
from jax.experimental import pallas as pl
from jax.experimental.pallas import tpu as pltpu
from jax.experimental.pallas import tpu_sc as plsc

_L = 16
_NW = 32
_ETA_PAD = 128


def _make_sc_kernel(n_atoms: int):
    n_groups = n_atoms // _L
    gp = -(-n_groups // _NW)
    last_groups = n_groups - (_NW - 1) * gp
    ch = gp * _L
    fw = 3 * ch
    last_ch = last_groups * _L
    last_fw = 3 * last_ch

    mesh = plsc.VectorSubcoreMesh(core_axis_name="c", subcore_axis_name="s")

    @functools.partial(
        pl.kernel,
        out_type=jax.ShapeDtypeStruct((3 * n_atoms,), jnp.float32),
        mesh=mesh,
        scratch_types=[
            pltpu.VMEM((ch,), jnp.int32),
            pltpu.VMEM((fw,), jnp.float32),
            pltpu.VMEM((_ETA_PAD,), jnp.float32),
        ],
        compiler_params=pltpu.CompilerParams(
            needs_layout_passes=False,
            skip_device_barrier=True,
            disable_bounds_checks=True,
            disable_semaphore_checks=True,
        ),
    )
    def body(f_hbm, z_hbm, eta_hbm, out_hbm, z_v, f_v, inv_v):
        wid = lax.axis_index("s") * 2 + lax.axis_index("c")
        base = wid * ch
        fbase = wid * fw

        pltpu.sync_copy(eta_hbm, inv_v)
        for i in range(_ETA_PAD // _L):
            sl = pl.ds(i * _L, _L)
            inv_v[sl] = 1.0 / inv_v[sl]

        @pl.when(wid < _NW - 1)
        def _():
            pltpu.sync_copy(z_hbm.at[pl.ds(base, ch)], z_v)
            pltpu.sync_copy(f_hbm.at[pl.ds(fbase, fw)], f_v)

        @pl.when(wid == _NW - 1)
        def _():
            pltpu.sync_copy(z_hbm.at[pl.ds(base, last_ch)],
                            z_v.at[pl.ds(0, last_ch)])
            pltpu.sync_copy(f_hbm.at[pl.ds(fbase, last_fw)],
                            f_v.at[pl.ds(0, last_fw)])

        iota = lax.iota(jnp.int32, _L)
        idxv = [lax.div(iota + _L * v, 3) for v in range(3)]

        def group(g, carry):
            a16 = g * _L
            fb = g * (3 * _L)
            for v in range(3):
                zg = plsc.load_gather(z_v, [a16 + idxv[v]])
                r = plsc.load_gather(inv_v, [jnp.bitwise_and(zg, _ETA_PAD - 1)])
                sl = pl.ds(fb + v * _L, _L)
                f_v[sl] = f_v[sl] * r
            return carry

        lax.fori_loop(0, gp, group, 0, unroll=2)

        @pl.when(wid < _NW - 1)
        def _():
            pltpu.sync_copy(f_v, out_hbm.at[pl.ds(fbase, fw)])

        @pl.when(wid == _NW - 1)
        def _():
            pltpu.sync_copy(f_v.at[pl.ds(0, last_fw)],
                            out_hbm.at[pl.ds(fbase, last_fw)])

    return body


def kernel(forces, Z, eta):
    n = forces.shape[0]
    f_flat = forces.reshape(-1)
    z = Z.astype(jnp.int32)
    eta_p = jnp.concatenate(
        [eta, jnp.ones((_ETA_PAD - eta.shape[0],), jnp.float32)])
    out = _make_sc_kernel(n)(f_flat, z, eta_p)
    return out.reshape(n, 3)


# speedup vs baseline: 1.4169x; 1.4169x over previous
import functools

import jax
import jax.numpy as jnp
from jax import lax
from jax.experimental import pallas as pl
from jax.experimental.pallas import tpu as pltpu
from jax.experimental.pallas import tpu_sc as plsc

_L = 16
_NW = 32
_ETA_PAD = 128
_CHUNK = 448


def _make_sc_kernel(n_atoms: int):
    n_groups = n_atoms // _L
    gp = -(-n_groups // _NW)
    ch = gp * _L
    last_ch = n_atoms - (_NW - 1) * ch
    nfull_last = last_ch // _CHUNK
    rem = last_ch - nfull_last * _CHUNK
    assert ch % _CHUNK == 0 and rem % _L == 0

    mesh = plsc.VectorSubcoreMesh(core_axis_name="c", subcore_axis_name="s")

    @functools.partial(
        pl.kernel,
        out_type=jax.ShapeDtypeStruct((n_atoms, 3), jnp.float32),
        mesh=mesh,
        scratch_types=[
            pltpu.VMEM((_CHUNK,), jnp.int32),
            pltpu.VMEM((_CHUNK, 3), jnp.float32),
            pltpu.VMEM((_ETA_PAD,), jnp.float32),
        ],
        compiler_params=pltpu.CompilerParams(needs_layout_passes=False),
    )
    def body(f_hbm, z_hbm, eta_hbm, out_hbm, z_v, f_v, inv_v):
        wid = lax.axis_index("s") * 2 + lax.axis_index("c")
        base = wid * ch

        pltpu.sync_copy(eta_hbm, inv_v)
        for i in range(_ETA_PAD // _L):
            sl = pl.ds(i * _L, _L)
            inv_v[sl] = 1.0 / inv_v[sl]

        iota = lax.iota(jnp.int32, _L)
        rows = [lax.div(iota + _L * v, 3) for v in range(3)]
        cols = [lax.rem(iota + _L * v, 3) for v in range(3)]

        def compute(groups):
            def group(g, carry):
                a16 = g * _L
                for v in range(3):
                    rv = a16 + rows[v]
                    zg = plsc.load_gather(z_v, [rv])
                    r = plsc.load_gather(
                        inv_v, [jnp.bitwise_and(zg, _ETA_PAD - 1)])
                    fv = plsc.load_gather(f_v, [rv, cols[v]])
                    plsc.store_scatter(f_v, [rv, cols[v]], fv * r)
                return carry
            lax.fori_loop(0, groups, group, 0, unroll=2)

        def chunk(k, carry):
            off = base + k * _CHUNK
            pltpu.sync_copy(z_hbm.at[pl.ds(off, _CHUNK)], z_v)
            pltpu.sync_copy(f_hbm.at[pl.ds(off, _CHUNK)], f_v)
            compute(_CHUNK // _L)
            pltpu.sync_copy(f_v, out_hbm.at[pl.ds(off, _CHUNK)])
            return carry

        nfull = jnp.where(wid == _NW - 1, nfull_last, ch // _CHUNK)
        lax.fori_loop(0, nfull, chunk, 0)

        if rem:
            @pl.when(wid == _NW - 1)
            def _():
                off = base + nfull_last * _CHUNK
                pltpu.sync_copy(z_hbm.at[pl.ds(off, rem)],
                                z_v.at[pl.ds(0, rem)])
                pltpu.sync_copy(f_hbm.at[pl.ds(off, rem)],
                                f_v.at[pl.ds(0, rem)])
                compute(rem // _L)
                pltpu.sync_copy(f_v.at[pl.ds(0, rem)],
                                out_hbm.at[pl.ds(off, rem)])

    return body


def kernel(forces, Z, eta):
    n = forces.shape[0]
    z = Z.astype(jnp.int32)
    eta_p = jnp.concatenate(
        [eta, jnp.ones((_ETA_PAD - eta.shape[0],), jnp.float32)])
    return _make_sc_kernel(n)(forces, z, eta_p)


# speedup vs baseline: 5.3205x; 3.7550x over previous
import functools

import jax
import jax.numpy as jnp
from jax import lax
from jax.experimental import pallas as pl
from jax.experimental.pallas import tpu as pltpu
from jax.experimental.pallas import tpu_sc as plsc

_L = 16
_NW = 32
_ETA_PAD = 128
_U = 128


def _make_sc_kernel(n_pad: int):
    units = n_pad // _U
    u_lo = units // _NW
    n_hi = units - u_lo * _NW
    ch_hi = (u_lo + 1) * _U
    ch_lo = u_lo * _U

    mesh = plsc.VectorSubcoreMesh(core_axis_name="c", subcore_axis_name="s")

    @functools.partial(
        pl.kernel,
        out_type=jax.ShapeDtypeStruct((3, n_pad), jnp.float32),
        mesh=mesh,
        scratch_types=[
            pltpu.VMEM((ch_hi,), jnp.int32),
            pltpu.VMEM((3, ch_hi), jnp.float32),
            pltpu.VMEM((_ETA_PAD,), jnp.float32),
        ],
        compiler_params=pltpu.CompilerParams(needs_layout_passes=False),
    )
    def body(f_hbm, z_hbm, eta_hbm, out_hbm, z_v, f_v, inv_v):
        wid = lax.axis_index("s") * 2 + lax.axis_index("c")
        base = _U * (wid * u_lo + jnp.minimum(wid, n_hi))

        pltpu.sync_copy(eta_hbm, inv_v)
        for i in range(_ETA_PAD // _L):
            sl = pl.ds(i * _L, _L)
            inv_v[sl] = 1.0 / inv_v[sl]

        @pl.when(wid < n_hi)
        def _():
            pltpu.sync_copy(z_hbm.at[pl.ds(base, ch_hi)], z_v)
            pltpu.sync_copy(f_hbm.at[:, pl.ds(base, ch_hi)], f_v)

        @pl.when(wid >= n_hi)
        def _():
            pltpu.sync_copy(z_hbm.at[pl.ds(base, ch_lo)],
                            z_v.at[pl.ds(0, ch_lo)])
            pltpu.sync_copy(f_hbm.at[:, pl.ds(base, ch_lo)],
                            f_v.at[:, pl.ds(0, ch_lo)])

        def group(g, carry):
            sl = pl.ds(g * _L, _L)
            z = z_v[sl]
            r = plsc.load_gather(inv_v, [jnp.bitwise_and(z, _ETA_PAD - 1)])
            for c in range(3):
                f_v[c, sl] = f_v[c, sl] * r
            return carry

        ng = jnp.where(wid < n_hi, ch_hi // _L, ch_lo // _L)
        lax.fori_loop(0, ng, group, 0)

        @pl.when(wid < n_hi)
        def _():
            pltpu.sync_copy(f_v, out_hbm.at[:, pl.ds(base, ch_hi)])

        @pl.when(wid >= n_hi)
        def _():
            pltpu.sync_copy(f_v.at[:, pl.ds(0, ch_lo)],
                            out_hbm.at[:, pl.ds(base, ch_lo)])

    return body


def kernel(forces, Z, eta):
    n = forces.shape[0]
    n_pad = -(-n // _U) * _U
    ft = jnp.pad(forces.T, ((0, 0), (0, n_pad - n)))
    z = jnp.pad(Z.astype(jnp.int32), (0, n_pad - n))
    eta_p = jnp.concatenate(
        [eta, jnp.ones((_ETA_PAD - eta.shape[0],), jnp.float32)])
    out = _make_sc_kernel(n_pad)(ft, z, eta_p)
    return out[:, :n].T


# speedup vs baseline: 5.9012x; 1.1091x over previous
import functools

import jax
import jax.numpy as jnp
from jax import lax
from jax.experimental import pallas as pl
from jax.experimental.pallas import tpu as pltpu
from jax.experimental.pallas import tpu_sc as plsc

_L = 16
_NW = 32
_U = 128
_NE = 119


def _n_pad(n: int) -> int:
    units = -(-n // _U)
    return -(-units // _NW) * _NW * _U


def _make_sc_kernel(n: int):
    n_pad = _n_pad(n)
    ch = n_pad // _NW
    last_base = (_NW - 1) * ch
    z_full = ((n - last_base) // _U) * _U
    z_rem = n - last_base - z_full
    assert z_rem % 8 == 0

    mesh = plsc.VectorSubcoreMesh(core_axis_name="c", subcore_axis_name="s")

    @functools.partial(
        pl.kernel,
        out_type=jax.ShapeDtypeStruct((3, n_pad), jnp.float32),
        mesh=mesh,
        scratch_types=[
            pltpu.VMEM((ch,), jnp.int32),
            pltpu.VMEM((3, ch), jnp.float32),
            pltpu.VMEM((_NE,), jnp.float32),
            pltpu.VMEM((_NE,), jnp.float32),
            pltpu.SemaphoreType.DMA,
        ],
        compiler_params=pltpu.CompilerParams(needs_layout_passes=False),
    )
    def body(f_hbm, z_hbm, eta_hbm, out_hbm, z_v, f_v, eta_v, inv_v, sem_f):
        wid = lax.axis_index("s") * 2 + lax.axis_index("c")
        base = wid * ch

        f_in = pltpu.async_copy(f_hbm.at[:, pl.ds(base, ch)], f_v, sem_f)

        @pl.when(wid < _NW - 1)
        def _():
            pltpu.sync_copy(z_hbm.at[pl.ds(base, ch)], z_v)

        @pl.when(wid == _NW - 1)
        def _():
            pltpu.sync_copy(z_hbm.at[pl.ds(base, z_full)],
                            z_v.at[pl.ds(0, z_full)])
            if z_rem:
                pltpu.sync_copy(z_hbm.at[pl.ds(base + z_full, z_rem)],
                                z_v.at[pl.ds(z_full, z_rem)])

        pltpu.sync_copy(eta_hbm, eta_v)
        for i in range(8):
            sl = pl.ds(min(i * _L, _NE - _L), _L)
            inv_v[sl] = 1.0 / eta_v[sl]

        f_in.wait()

        def group(g, carry):
            sl = pl.ds(g * _L, _L)
            r = plsc.load_gather(inv_v, [jnp.minimum(z_v[sl], _NE - 1)])
            for c in range(3):
                f_v[c, sl] = f_v[c, sl] * r
            return carry

        lax.fori_loop(0, ch // _L, group, 0, unroll=4)

        pltpu.sync_copy(f_v, out_hbm.at[:, pl.ds(base, ch)])

    return body


def kernel(forces, Z, eta):
    n = forces.shape[0]
    ft = jnp.pad(forces.T, ((0, 0), (0, _n_pad(n) - n)))
    out = _make_sc_kernel(n)(ft, Z.astype(jnp.int32), eta)
    return out[:, :n].T


# speedup vs baseline: 6.0701x; 1.0286x over previous
import functools

import jax
import jax.numpy as jnp
from jax import lax
from jax.experimental import pallas as pl
from jax.experimental.pallas import tpu as pltpu
from jax.experimental.pallas import tpu_sc as plsc

_L = 16
_NW = 32
_U = 128
_NE = 119


def _n_pad(n: int) -> int:
    units = -(-n // _U)
    return -(-units // _NW) * _NW * _U


def _make_sc_kernel(n: int):
    n_pad = _n_pad(n)
    ch = n_pad // _NW
    last_base = (_NW - 1) * ch
    z_full = ((n - last_base) // _U) * _U
    z_rem = n - last_base - z_full
    assert z_rem % 8 == 0

    mesh = plsc.VectorSubcoreMesh(core_axis_name="c", subcore_axis_name="s")

    @functools.partial(
        pl.kernel,
        out_type=jax.ShapeDtypeStruct((3, n_pad), jnp.float32),
        mesh=mesh,
        scratch_types=[
            pltpu.VMEM((ch,), jnp.int32),
            pltpu.VMEM((3, ch), jnp.float32),
            pltpu.VMEM((_NE,), jnp.float32),
            pltpu.VMEM((_NE,), jnp.float32),
            pltpu.SemaphoreType.DMA,
        ],
        compiler_params=pltpu.CompilerParams(needs_layout_passes=False),
    )
    def body(f_hbm, z_hbm, eta_hbm, out_hbm, z_v, f_v, eta_v, inv_v, sem_f):
        wid = lax.axis_index("s") * 2 + lax.axis_index("c")
        base = wid * ch

        f_in = pltpu.async_copy(f_hbm.at[:, pl.ds(base, ch)], f_v, sem_f)

        @pl.when(wid < _NW - 1)
        def _():
            pltpu.sync_copy(z_hbm.at[pl.ds(base, ch)], z_v)

        @pl.when(wid == _NW - 1)
        def _():
            pltpu.sync_copy(z_hbm.at[pl.ds(base, z_full)],
                            z_v.at[pl.ds(0, z_full)])
            if z_rem:
                pltpu.sync_copy(z_hbm.at[pl.ds(base + z_full, z_rem)],
                                z_v.at[pl.ds(z_full, z_rem)])

        pltpu.sync_copy(eta_hbm, eta_v)
        for i in range(8):
            sl = pl.ds(min(i * _L, _NE - _L), _L)
            inv_v[sl] = 1.0 / eta_v[sl]

        f_in.wait()

        def group(g, carry):
            sl = pl.ds(g * _L, _L)
            r = plsc.load_gather(inv_v, [jnp.minimum(z_v[sl], _NE - 1)])
            for c in range(3):
                f_v[c, sl] = f_v[c, sl] * r
            return carry

        lax.fori_loop(0, ch // _L, group, 0, unroll=8)

        pltpu.sync_copy(f_v, out_hbm.at[:, pl.ds(base, ch)])

    return body


def kernel(forces, Z, eta):
    n = forces.shape[0]
    ft = jnp.pad(forces.T, ((0, 0), (0, _n_pad(n) - n)))
    out = _make_sc_kernel(n)(ft, Z.astype(jnp.int32), eta)
    return out[:, :n].T
